# baseline (speedup 1.0000x reference)
import functools

import jax
import jax.numpy as jnp
from jax import lax
from jax.experimental import pallas as pl
from jax.experimental.pallas import tpu as pltpu
from jax.experimental.pallas import tpu_sc as plsc

NUM_POS_FEATS = 256
HALF = NUM_POS_FEATS // 2
LANES = 16


def _make_sc_kernel(b, h, w):
    info = plsc.get_sparse_core_info()
    nc, ns = info.num_cores, info.num_subcores
    nw = nc * ns
    assert h % nw == 0
    rpw = h // nw
    mesh = plsc.VectorSubcoreMesh(core_axis_name="c", subcore_axis_name="s")

    @functools.partial(
        pl.kernel,
        mesh=mesh,
        out_type=jax.ShapeDtypeStruct((b, h, w, NUM_POS_FEATS), jnp.float32),
        scratch_types=[
            pltpu.VMEM((w, HALF), jnp.float32),
            pltpu.VMEM((h, HALF), jnp.float32),
            pltpu.VMEM((rpw, w, NUM_POS_FEATS), jnp.float32),
            pltpu.VMEM_SHARED((ns, w, NUM_POS_FEATS), jnp.float32),
            pltpu.SemaphoreType.DMA,
            pltpu.SemaphoreType.DMA,
        ],
    )
    def sc_kernel(col_hbm, row_hbm, out_hbm, col_v, row_v, chunk, shared, sem, sem2):
        wid = lax.axis_index("s") * nc + lax.axis_index("c")
        i0 = wid * rpw
        pltpu.sync_copy(col_hbm.at[pl.ds(0, w)], col_v)
        pltpu.sync_copy(row_hbm.at[pl.ds(0, h)], row_v)
        rv = [
            [row_v[i0 + r, pl.ds(LANES * k, LANES)] for k in range(HALF // LANES)]
            for r in range(rpw)
        ]

        def body(j, carry):
            for k in range(HALF // LANES):
                cv = col_v[j, pl.ds(LANES * k, LANES)]
                for r in range(rpw):
                    chunk[r, j, pl.ds(LANES * k, LANES)] = cv
            for r in range(rpw):
                for k in range(HALF // LANES):
                    chunk[r, j, pl.ds(HALF + LANES * k, LANES)] = rv[r][k]
            return carry

        lax.fori_loop(0, w, body, 0)

        sid = lax.axis_index("s")
        copies = [
            pltpu.async_copy(
                chunk.at[pl.ds(0, rpw - 1)], out_hbm.at[bb, pl.ds(i0, rpw - 1)], sem
            )
            for bb in range(b)
        ]
        pltpu.sync_copy(chunk.at[rpw - 1], shared.at[sid])
        copies2 = [
            pltpu.async_copy(shared.at[sid], out_hbm.at[bb, i0 + rpw - 1], sem2)
            for bb in range(b)
        ]
        for c in copies:
            c.wait()
        for c in copies2:
            c.wait()

    return sc_kernel


def kernel(tensor_list, row_embed, col_embed):
    b, h, w = tensor_list.shape[0], tensor_list.shape[-3], tensor_list.shape[-2]
    return _make_sc_kernel(b, h, w)(col_embed, row_embed)


# speedup vs baseline: 1.3041x; 1.3041x over previous
import jax
import jax.numpy as jnp
from jax.experimental import pallas as pl
from jax.experimental.pallas import tpu as pltpu

NUM_POS_FEATS = 256
NSEM = 4


def _make_body(b, h, w):
    half = NUM_POS_FEATS // 2

    def _body(col_ref, row_ref, out_ref, scratch, *sems):
        col = col_ref[:w, :]
        row = row_ref[:h, :]
        scratch[:, :, :half] = jnp.broadcast_to(col[None, :, :], (h, w, half))
        scratch[:, :, half:] = jnp.broadcast_to(row[:, None, :], (h, w, half))
        copies = [
            pltpu.make_async_copy(scratch, out_ref.at[i], sems[i % NSEM])
            for i in range(b)
        ]
        for c in copies:
            c.start()
        for c in copies:
            c.wait()

    return _body


def kernel(tensor_list, row_embed, col_embed):
    b, h, w = tensor_list.shape[0], tensor_list.shape[-3], tensor_list.shape[-2]
    out = pl.pallas_call(
        _make_body(b, h, w),
        in_specs=[
            pl.BlockSpec(memory_space=pltpu.VMEM),
            pl.BlockSpec(memory_space=pltpu.VMEM),
        ],
        out_specs=pl.BlockSpec(memory_space=pl.ANY),
        out_shape=jax.ShapeDtypeStruct((b, h, w, NUM_POS_FEATS), jnp.float32),
        scratch_shapes=[
            pltpu.VMEM((h, w, NUM_POS_FEATS), jnp.float32),
        ]
        + [pltpu.SemaphoreType.DMA] * NSEM,
    )(col_embed, row_embed)
    return out
